# Initial kernel scaffold; baseline (speedup 1.0000x reference)
import jax
import jax.numpy as jnp
from jax.experimental import pallas as pl


def kernel(patch_logits):
    raise NotImplementedError("write your pallas kernel here")


# speedup vs baseline: 18.1510x; 18.1510x over previous
import functools

import jax
import jax.numpy as jnp
from jax.experimental import pallas as pl


def _quantile_pool_body(x_ref, out_ref, *, k_above, n_above_lo):
    x = x_ref[...]
    bits = jax.lax.bitcast_convert_type(x, jnp.int32)
    key = bits ^ (jax.lax.shift_right_arithmetic(bits, 31) & jnp.int32(0x7FFFFFFF))

    int_min = jnp.int32(-(2 ** 31))
    t0 = jnp.full((x.shape[0], 1), int_min, dtype=jnp.int32)

    def step(i, t):
        shift = jnp.left_shift(jnp.int32(1), jnp.int32(31) - i)
        cand = t + shift
        cnt = jnp.sum((key >= cand).astype(jnp.int32), axis=1, keepdims=True)
        return jnp.where(cnt >= k_above, cand, t)

    t = jax.lax.fori_loop(0, 32, step, t0)

    ge = key >= t
    gt = key > t
    c_ge = jnp.sum(ge.astype(jnp.int32), axis=1, keepdims=True)
    c_gt = jnp.sum(gt.astype(jnp.int32), axis=1, keepdims=True)
    s_ge = jnp.sum(jnp.where(ge, x, 0.0), axis=1, keepdims=True)
    s_gt = jnp.sum(jnp.where(gt, x, 0.0), axis=1, keepdims=True)

    dup = c_ge >= n_above_lo
    cnt = jnp.where(dup, c_gt, c_ge).astype(jnp.float32)
    s = jnp.where(dup, s_gt, s_ge)
    out_ref[...] = s / cnt


@jax.jit
def kernel(patch_logits):
    b, n = patch_logits.shape
    q = (100 - 10) / 100.0
    i0 = int(q * (n - 1))
    k_above = n - i0 - 1
    rows_per_block = 16
    grid = b // rows_per_block

    return pl.pallas_call(
        functools.partial(_quantile_pool_body, k_above=k_above,
                          n_above_lo=n - i0),
        grid=(grid,),
        in_specs=[pl.BlockSpec((rows_per_block, n), lambda i: (i, 0))],
        out_specs=pl.BlockSpec((rows_per_block, 1), lambda i: (i, 0)),
        out_shape=jax.ShapeDtypeStruct((b, 1), jnp.float32),
    )(patch_logits)
